# Initial kernel scaffold; baseline (speedup 1.0000x reference)
import jax
import jax.numpy as jnp
from jax.experimental import pallas as pl


def kernel(x, edge_index, inp_params, edge_params, node_params):
    raise NotImplementedError("write your pallas kernel here")


# speedup vs baseline: 1.6636x; 1.6636x over previous
import functools

import jax
import jax.numpy as jnp
from jax import lax
from jax.experimental import pallas as pl
from jax.experimental.pallas import tpu as pltpu
from jax.experimental.pallas import tpu_sc as plsc

N = 10000
E = 160000
EP = 163840
D = 80
LAT = 67
HID = 64
NC = 2
NS = 16
NW = NC * NS
EW = EP // NW
CH = 128
CHUNKS = EW // CH
GRP = 4
GROUP_ROWS = GRP * CH
NGROUPS = CHUNKS // GRP
GRP_SC = 2
GROUP_ROWS_SC = GRP_SC * CH
NGROUPS_SC = CHUNKS // GRP_SC
NODE_ROWS_PER_TILE = N // NS

R_EDGE = 640
R_NODE = 1000

@functools.cache
def _sc_mesh():
    return plsc.VectorSubcoreMesh(core_axis_name="c", subcore_axis_name="s",
                                  num_cores=NC, num_subcores=NS)


def _sc_gather(h, s2d, e2d):
    def body(h_hbm, s_hbm, e_hbm, os_hbm, oe_hbm,
             sidx, eidx, buf_a, buf_b, sem_a, sem_b):
        wid = lax.axis_index("s") * NC + lax.axis_index("c")
        pltpu.sync_copy(s_hbm.at[pl.ds(wid * CHUNKS, CHUNKS)], sidx)
        pltpu.sync_copy(e_hbm.at[pl.ds(wid * CHUNKS, CHUNKS)], eidx)

        def group(g, carry):
            gbase = wid * EW + g * GROUP_ROWS
            for k in range(GRP):
                pltpu.async_copy(h_hbm.at[sidx.at[g * GRP + k]],
                                 buf_a.at[pl.ds(k * CH, CH)], sem_a)
            for k in range(GRP):
                pltpu.async_copy(h_hbm.at[eidx.at[g * GRP + k]],
                                 buf_b.at[pl.ds(k * CH, CH)], sem_b)
            pltpu.make_async_copy(os_hbm.at[pl.ds(0, GROUP_ROWS)], buf_a, sem_a).wait()
            pltpu.sync_copy(buf_a, os_hbm.at[pl.ds(gbase, GROUP_ROWS)])
            pltpu.make_async_copy(oe_hbm.at[pl.ds(0, GROUP_ROWS)], buf_b, sem_b).wait()
            pltpu.sync_copy(buf_b, oe_hbm.at[pl.ds(gbase, GROUP_ROWS)])
            return carry

        lax.fori_loop(0, NGROUPS, group, 0)

    call = pl.kernel(
        body,
        out_type=(jax.ShapeDtypeStruct((EP, D), jnp.float32),
                  jax.ShapeDtypeStruct((EP, D), jnp.float32)),
        mesh=_sc_mesh(),
        scratch_types=(
            pltpu.VMEM((CHUNKS, CH), jnp.int32),
            pltpu.VMEM((CHUNKS, CH), jnp.int32),
            pltpu.VMEM((GROUP_ROWS, D), jnp.float32),
            pltpu.VMEM((GROUP_ROWS, D), jnp.float32),
            pltpu.SemaphoreType.DMA,
            pltpu.SemaphoreType.DMA,
        ),
        compiler_params=pltpu.CompilerParams(use_tc_tiling_on_sc=False),
    )
    return call(h, s2d, e2d)


def _sc_scatter(ws, we, s2d, e2d, zeros_nd):
    def body(ws_hbm, we_hbm, s_hbm, e_hbm, z_hbm, out_hbm,
             sidx, eidx, vals_a, vals_b, sem_a, sem_b, accum):
        cid = lax.axis_index("c")
        sid = lax.axis_index("s")
        wid = sid * NC + cid
        pltpu.sync_copy(s_hbm.at[pl.ds(wid * CHUNKS, CHUNKS)], sidx)
        pltpu.sync_copy(e_hbm.at[pl.ds(wid * CHUNKS, CHUNKS)], eidx)

        @pl.when(sid == 0)
        def _():
            pltpu.sync_copy(z_hbm, accum)

        plsc.subcore_barrier()

        def group(g, carry):
            gbase = wid * EW + g * GROUP_ROWS_SC
            pltpu.async_copy(ws_hbm.at[pl.ds(gbase, GROUP_ROWS_SC)], vals_a, sem_a)
            pltpu.async_copy(we_hbm.at[pl.ds(gbase, GROUP_ROWS_SC)], vals_b, sem_b)
            pltpu.make_async_copy(ws_hbm.at[pl.ds(0, GROUP_ROWS_SC)], vals_a, sem_a).wait()
            for k in range(GRP_SC):
                pltpu.sync_copy(vals_a.at[pl.ds(k * CH, CH)],
                                accum.at[eidx.at[g * GRP_SC + k]], add=True)
            pltpu.make_async_copy(we_hbm.at[pl.ds(0, GROUP_ROWS_SC)], vals_b, sem_b).wait()
            for k in range(GRP_SC):
                pltpu.sync_copy(vals_b.at[pl.ds(k * CH, CH)],
                                accum.at[sidx.at[g * GRP_SC + k]], add=True)
            return carry

        lax.fori_loop(0, NGROUPS_SC, group, 0)

        plsc.subcore_barrier()
        pltpu.sync_copy(accum.at[pl.ds(sid * NODE_ROWS_PER_TILE, NODE_ROWS_PER_TILE)],
                        out_hbm.at[cid, pl.ds(sid * NODE_ROWS_PER_TILE, NODE_ROWS_PER_TILE)])

    call = pl.kernel(
        body,
        out_type=jax.ShapeDtypeStruct((NC, N, D), jnp.float32),
        mesh=_sc_mesh(),
        scratch_types=(
            pltpu.VMEM((CHUNKS, CH), jnp.int32),
            pltpu.VMEM((CHUNKS, CH), jnp.int32),
            pltpu.VMEM((GROUP_ROWS_SC, D), jnp.float32),
            pltpu.VMEM((GROUP_ROWS_SC, D), jnp.float32),
            pltpu.SemaphoreType.DMA,
            pltpu.SemaphoreType.DMA,
            pltpu.VMEM_SHARED((N, D), jnp.float32),
        ),
        compiler_params=pltpu.CompilerParams(use_tc_tiling_on_sc=False),
    )
    return call(ws, we, s2d, e2d, zeros_nd)


def _ln_tanh(z, g, beta, nreal):
    mu = jnp.sum(z, axis=1, keepdims=True) * (1.0 / nreal)
    msq = jnp.sum(z * z, axis=1, keepdims=True) * (1.0 / nreal)
    var = msq - mu * mu
    return jnp.tanh((z - mu) * jax.lax.rsqrt(var + 1e-5) * g + beta)


def _dot(a, b):
    return jnp.dot(a, b, preferred_element_type=jnp.float32)


def _full_spec(shape):
    return pl.BlockSpec(shape, lambda i: tuple(0 for _ in shape))


def _row_spec(rows, cols):
    return pl.BlockSpec((rows, cols), lambda i: (i, 0))


def _edge_mlp(hs, he, w):
    t = _ln_tanh(_dot(hs, w["Wa"]) + _dot(he, w["Wb"]) + w["b1"], w["g1"], w["be1"], LAT)
    t = _ln_tanh(_dot(t, w["W2"]) + w["b2"], w["g2"], w["be2"], LAT)
    t = _ln_tanh(_dot(t, w["W3"]) + w["b3"], w["g3"], w["be3"], LAT)
    return jnp.sum(t * w["w4"], axis=1, keepdims=True) + w["b4"]


_EDGE_W_NAMES = ("Wa", "Wb", "b1", "g1", "be1", "W2", "b2", "g2", "be2",
                 "W3", "b3", "g3", "be3", "w4", "b4")


def _tc_edge_messages(hs, he, ew):
    def body(hs_ref, he_ref, *rest):
        wrefs = rest[:len(_EDGE_W_NAMES)]
        ws_ref, we_ref = rest[len(_EDGE_W_NAMES):]
        w = {n: r[...] for n, r in zip(_EDGE_W_NAMES, wrefs)}
        a = hs_ref[...]
        b = he_ref[...]
        logit = _edge_mlp(a, b, w)
        s = 1.0 / (1.0 + jnp.exp(-logit))
        rid = (pl.program_id(0) * R_EDGE
               + lax.broadcasted_iota(jnp.int32, (R_EDGE, 1), 0))
        s = jnp.where(rid < E, s, 0.0)
        ws_ref[...] = s * a
        we_ref[...] = s * b

    wvals = [ew[n] for n in _EDGE_W_NAMES]
    return pl.pallas_call(
        body,
        grid=(EP // R_EDGE,),
        in_specs=[_row_spec(R_EDGE, D), _row_spec(R_EDGE, D)]
                 + [_full_spec(v.shape) for v in wvals],
        out_specs=[_row_spec(R_EDGE, D), _row_spec(R_EDGE, D)],
        out_shape=[jax.ShapeDtypeStruct((EP, D), jnp.float32),
                   jax.ShapeDtypeStruct((EP, D), jnp.float32)],
    )(hs, he, *wvals)


def _tc_edge_logits(hs, he, ew):
    def body(hs_ref, he_ref, *rest):
        wrefs = rest[:len(_EDGE_W_NAMES)]
        out_ref = rest[len(_EDGE_W_NAMES)]
        w = {n: r[...] for n, r in zip(_EDGE_W_NAMES, wrefs)}
        out_ref[...] = _edge_mlp(hs_ref[...], he_ref[...], w)

    wvals = [ew[n] for n in _EDGE_W_NAMES]
    return pl.pallas_call(
        body,
        grid=(EP // R_EDGE,),
        in_specs=[_row_spec(R_EDGE, D), _row_spec(R_EDGE, D)]
                 + [_full_spec(v.shape) for v in wvals],
        out_specs=_row_spec(R_EDGE, 1),
        out_shape=jax.ShapeDtypeStruct((EP, 1), jnp.float32),
    )(hs, he, *wvals)


_NODE_W_NAMES = ("Va", "Vb", "c1", "g1", "be1", "V2", "c2", "g2", "be2",
                 "V3", "c3")


def _tc_node(m0, m1, h, xpad, nw):
    def body(m0_ref, m1_ref, h_ref, xp_ref, *rest):
        wrefs = rest[:len(_NODE_W_NAMES)]
        out_ref = rest[len(_NODE_W_NAMES)]
        w = {n: r[...] for n, r in zip(_NODE_W_NAMES, wrefs)}
        m = m0_ref[...] + m1_ref[...]
        hh = h_ref[...]
        t = _ln_tanh(_dot(m, w["Va"]) + _dot(hh, w["Vb"]) + w["c1"],
                     w["g1"], w["be1"], HID)
        t = _ln_tanh(_dot(t, w["V2"]) + w["c2"], w["g2"], w["be2"], HID)
        t = _dot(t, w["V3"]) + w["c3"]
        out_ref[...] = hh + t + xp_ref[...]

    wvals = [nw[n] for n in _NODE_W_NAMES]
    return pl.pallas_call(
        body,
        grid=(N // R_NODE,),
        in_specs=[_row_spec(R_NODE, D)] * 4 + [_full_spec(v.shape) for v in wvals],
        out_specs=_row_spec(R_NODE, D),
        out_shape=jax.ShapeDtypeStruct((N, D), jnp.float32),
    )(m0, m1, h, xpad, *wvals)


_INP_W_NAMES = ("U1", "d1", "g1", "be1", "U2", "d2", "g2", "be2",
                "U3", "d3", "g3", "be3")


def _tc_input(x0, xpad, iw):
    def body(x0_ref, xp_ref, *rest):
        wrefs = rest[:len(_INP_W_NAMES)]
        out_ref = rest[len(_INP_W_NAMES)]
        w = {n: r[...] for n, r in zip(_INP_W_NAMES, wrefs)}
        t = _ln_tanh(_dot(x0_ref[...], w["U1"]) + w["d1"], w["g1"], w["be1"], HID)
        t = _ln_tanh(_dot(t, w["U2"]) + w["d2"], w["g2"], w["be2"], HID)
        t = _ln_tanh(_dot(t, w["U3"]) + w["d3"], w["g3"], w["be3"], HID)
        out_ref[...] = t + xp_ref[...]

    wvals = [iw[n] for n in _INP_W_NAMES]
    return pl.pallas_call(
        body,
        grid=(N // R_NODE,),
        in_specs=[_row_spec(R_NODE, D)] * 2 + [_full_spec(v.shape) for v in wvals],
        out_specs=_row_spec(R_NODE, D),
        out_shape=jax.ShapeDtypeStruct((N, D), jnp.float32),
    )(x0, xpad, *wvals)


def _pad2(W):
    return jnp.zeros((D, D), jnp.float32).at[:W.shape[0], :W.shape[1]].set(W)


def _pad1(v):
    return jnp.zeros((1, D), jnp.float32).at[0, :v.shape[0]].set(v)


def _prep_edge_weights(edge_params):
    (W1, b1, g1, be1), (W2, b2, g2, be2), (W3, b3, g3, be3), (W4, b4) = edge_params
    return {
        "Wa": _pad2(W1[:LAT]), "Wb": _pad2(W1[LAT:]),
        "b1": _pad1(b1), "g1": _pad1(g1), "be1": _pad1(be1),
        "W2": _pad2(W2), "b2": _pad1(b2), "g2": _pad1(g2), "be2": _pad1(be2),
        "W3": _pad2(W3), "b3": _pad1(b3), "g3": _pad1(g3), "be3": _pad1(be3),
        "w4": _pad1(W4[:, 0]), "b4": b4.reshape(1, 1).astype(jnp.float32),
    }


def _prep_node_weights(node_params):
    (V1, c1, g1, be1), (V2, c2, g2, be2), (V3, c3) = node_params
    return {
        "Va": _pad2(V1[:LAT]), "Vb": _pad2(V1[LAT:]),
        "c1": _pad1(c1), "g1": _pad1(g1), "be1": _pad1(be1),
        "V2": _pad2(V2), "c2": _pad1(c2), "g2": _pad1(g2), "be2": _pad1(be2),
        "V3": _pad2(V3), "c3": _pad1(c3),
    }


def _prep_inp_weights(inp_params):
    (U1, d1, g1, be1), (U2, d2, g2, be2), (U3, d3, g3, be3) = inp_params
    return {
        "U1": _pad2(U1), "d1": _pad1(d1), "g1": _pad1(g1), "be1": _pad1(be1),
        "U2": _pad2(U2), "d2": _pad1(d2), "g2": _pad1(g2), "be2": _pad1(be2),
        "U3": _pad2(U3), "d3": _pad1(d3), "g3": _pad1(g3), "be3": _pad1(be3),
    }


def kernel(x, edge_index, inp_params, edge_params, node_params):
    start = edge_index[0].astype(jnp.int32)
    end = edge_index[1].astype(jnp.int32)
    s2d = jnp.zeros((EP,), jnp.int32).at[:E].set(start).reshape(NW * CHUNKS, CH)
    e2d = jnp.zeros((EP,), jnp.int32).at[:E].set(end).reshape(NW * CHUNKS, CH)

    xpad = jnp.zeros((N, D), jnp.float32).at[:, HID:LAT].set(x)
    x0 = jnp.zeros((N, D), jnp.float32).at[:, :x.shape[1]].set(x)
    zeros_nd = jnp.zeros((N, D), jnp.float32)

    iw = _prep_inp_weights(inp_params)
    ew = _prep_edge_weights(edge_params)
    nw = _prep_node_weights(node_params)

    h = _tc_input(x0, xpad, iw)
    for _ in range(8):
        hs, he = _sc_gather(h, s2d, e2d)
        ws, we = _tc_edge_messages(hs, he, ew)
        mp = _sc_scatter(ws, we, s2d, e2d, zeros_nd)
        h = _tc_node(mp[0], mp[1], h, xpad, nw)

    hs, he = _sc_gather(h, s2d, e2d)
    logits = _tc_edge_logits(hs, he, ew)
    return logits[:E, 0]


# speedup vs baseline: 1.6837x; 1.0121x over previous
import functools

import jax
import jax.numpy as jnp
from jax import lax
from jax.experimental import pallas as pl
from jax.experimental.pallas import tpu as pltpu
from jax.experimental.pallas import tpu_sc as plsc

N = 10000
E = 160000
EP = 163840
D = 80
LAT = 67
HID = 64
NC = 2
NS = 16
NW = NC * NS
EW = EP // NW
CH = 128
CHUNKS = EW // CH
GRP_G = 2
GR_G = GRP_G * CH
NGROUPS_G = CHUNKS // GRP_G
NPAIR_G = NGROUPS_G // 2
GRP_SC = 2
GROUP_ROWS_SC = GRP_SC * CH
NGROUPS_SC = CHUNKS // GRP_SC
NODE_ROWS_PER_TILE = N // NS

R_EDGE = 640
R_NODE = 1000

@functools.cache
def _sc_mesh():
    return plsc.VectorSubcoreMesh(core_axis_name="c", subcore_axis_name="s",
                                  num_cores=NC, num_subcores=NS)


def _sc_gather(h, s2d, e2d):
    def body(h_hbm, s_hbm, e_hbm, os_hbm, oe_hbm,
             sidx, eidx, a0, b0, a1, b1, sg0, sg1, sw0, sw1):
        wid = lax.axis_index("s") * NC + lax.axis_index("c")
        pltpu.sync_copy(s_hbm.at[pl.ds(wid * CHUNKS, CHUNKS)], sidx)
        pltpu.sync_copy(e_hbm.at[pl.ds(wid * CHUNKS, CHUNKS)], eidx)

        bufs = ((a0, b0, sg0, sw0), (a1, b1, sg1, sw1))

        def fire_set(k, g):
            ba, bb, sg, _ = bufs[k]
            for c in range(GRP_G):
                pltpu.async_copy(h_hbm.at[sidx.at[g * GRP_G + c]],
                                 ba.at[pl.ds(c * CH, CH)], sg)
            for c in range(GRP_G):
                pltpu.async_copy(h_hbm.at[eidx.at[g * GRP_G + c]],
                                 bb.at[pl.ds(c * CH, CH)], sg)

        def wait_gathers(k):
            ba, bb, sg, _ = bufs[k]
            pltpu.make_async_copy(os_hbm.at[pl.ds(0, GR_G)], ba, sg).wait()
            pltpu.make_async_copy(os_hbm.at[pl.ds(0, GR_G)], bb, sg).wait()

        def fire_writes(k, g):
            ba, bb, _, sw = bufs[k]
            gbase = wid * EW + g * GR_G
            pltpu.async_copy(ba, os_hbm.at[pl.ds(gbase, GR_G)], sw)
            pltpu.async_copy(bb, oe_hbm.at[pl.ds(gbase, GR_G)], sw)

        def wait_writes(k):
            ba, bb, _, sw = bufs[k]
            pltpu.make_async_copy(ba, os_hbm.at[pl.ds(0, GR_G)], sw).wait()
            pltpu.make_async_copy(bb, oe_hbm.at[pl.ds(0, GR_G)], sw).wait()

        fire_set(0, 0)

        def pair(i, carry):
            @pl.when(i > 0)
            def _():
                wait_writes(1)
            fire_set(1, 2 * i + 1)
            wait_gathers(0)
            fire_writes(0, 2 * i)
            wait_gathers(1)
            fire_writes(1, 2 * i + 1)
            wait_writes(0)

            @pl.when(i < NPAIR_G - 1)
            def _():
                fire_set(0, 2 * i + 2)
            return carry

        lax.fori_loop(0, NPAIR_G, pair, 0)
        wait_writes(1)

    call = pl.kernel(
        body,
        out_type=(jax.ShapeDtypeStruct((EP, D), jnp.float32),
                  jax.ShapeDtypeStruct((EP, D), jnp.float32)),
        mesh=_sc_mesh(),
        scratch_types=(
            pltpu.VMEM((CHUNKS, CH), jnp.int32),
            pltpu.VMEM((CHUNKS, CH), jnp.int32),
            pltpu.VMEM((GR_G, D), jnp.float32),
            pltpu.VMEM((GR_G, D), jnp.float32),
            pltpu.VMEM((GR_G, D), jnp.float32),
            pltpu.VMEM((GR_G, D), jnp.float32),
            pltpu.SemaphoreType.DMA,
            pltpu.SemaphoreType.DMA,
            pltpu.SemaphoreType.DMA,
            pltpu.SemaphoreType.DMA,
        ),
        compiler_params=pltpu.CompilerParams(use_tc_tiling_on_sc=False),
    )
    return call(h, s2d, e2d)


def _sc_scatter(ws, we, s2d, e2d, zeros_nd):
    def body(ws_hbm, we_hbm, s_hbm, e_hbm, z_hbm, out_hbm,
             sidx, eidx, vals_a, vals_b, sem_a, sem_b, accum):
        cid = lax.axis_index("c")
        sid = lax.axis_index("s")
        wid = sid * NC + cid
        pltpu.sync_copy(s_hbm.at[pl.ds(wid * CHUNKS, CHUNKS)], sidx)
        pltpu.sync_copy(e_hbm.at[pl.ds(wid * CHUNKS, CHUNKS)], eidx)

        @pl.when(sid == 0)
        def _():
            pltpu.sync_copy(z_hbm, accum)

        plsc.subcore_barrier()

        def group(g, carry):
            gbase = wid * EW + g * GROUP_ROWS_SC
            pltpu.async_copy(ws_hbm.at[pl.ds(gbase, GROUP_ROWS_SC)], vals_a, sem_a)
            pltpu.async_copy(we_hbm.at[pl.ds(gbase, GROUP_ROWS_SC)], vals_b, sem_b)
            pltpu.make_async_copy(ws_hbm.at[pl.ds(0, GROUP_ROWS_SC)], vals_a, sem_a).wait()
            for k in range(GRP_SC):
                pltpu.sync_copy(vals_a.at[pl.ds(k * CH, CH)],
                                accum.at[eidx.at[g * GRP_SC + k]], add=True)
            pltpu.make_async_copy(we_hbm.at[pl.ds(0, GROUP_ROWS_SC)], vals_b, sem_b).wait()
            for k in range(GRP_SC):
                pltpu.sync_copy(vals_b.at[pl.ds(k * CH, CH)],
                                accum.at[sidx.at[g * GRP_SC + k]], add=True)
            return carry

        lax.fori_loop(0, NGROUPS_SC, group, 0)

        plsc.subcore_barrier()
        pltpu.sync_copy(accum.at[pl.ds(sid * NODE_ROWS_PER_TILE, NODE_ROWS_PER_TILE)],
                        out_hbm.at[cid, pl.ds(sid * NODE_ROWS_PER_TILE, NODE_ROWS_PER_TILE)])

    call = pl.kernel(
        body,
        out_type=jax.ShapeDtypeStruct((NC, N, D), jnp.float32),
        mesh=_sc_mesh(),
        scratch_types=(
            pltpu.VMEM((CHUNKS, CH), jnp.int32),
            pltpu.VMEM((CHUNKS, CH), jnp.int32),
            pltpu.VMEM((GROUP_ROWS_SC, D), jnp.float32),
            pltpu.VMEM((GROUP_ROWS_SC, D), jnp.float32),
            pltpu.SemaphoreType.DMA,
            pltpu.SemaphoreType.DMA,
            pltpu.VMEM_SHARED((N, D), jnp.float32),
        ),
        compiler_params=pltpu.CompilerParams(use_tc_tiling_on_sc=False),
    )
    return call(ws, we, s2d, e2d, zeros_nd)


def _ln_tanh(z, g, beta, nreal):
    mu = jnp.sum(z, axis=1, keepdims=True) * (1.0 / nreal)
    msq = jnp.sum(z * z, axis=1, keepdims=True) * (1.0 / nreal)
    var = msq - mu * mu
    return jnp.tanh((z - mu) * jax.lax.rsqrt(var + 1e-5) * g + beta)


def _dot(a, b):
    return jnp.dot(a, b, preferred_element_type=jnp.float32)


def _full_spec(shape):
    return pl.BlockSpec(shape, lambda i: tuple(0 for _ in shape))


def _row_spec(rows, cols):
    return pl.BlockSpec((rows, cols), lambda i: (i, 0))


def _edge_mlp(hs, he, w):
    t = _ln_tanh(_dot(hs, w["Wa"]) + _dot(he, w["Wb"]) + w["b1"], w["g1"], w["be1"], LAT)
    t = _ln_tanh(_dot(t, w["W2"]) + w["b2"], w["g2"], w["be2"], LAT)
    t = _ln_tanh(_dot(t, w["W3"]) + w["b3"], w["g3"], w["be3"], LAT)
    return jnp.sum(t * w["w4"], axis=1, keepdims=True) + w["b4"]


_EDGE_W_NAMES = ("Wa", "Wb", "b1", "g1", "be1", "W2", "b2", "g2", "be2",
                 "W3", "b3", "g3", "be3", "w4", "b4")


def _tc_edge_messages(hs, he, ew):
    def body(hs_ref, he_ref, *rest):
        wrefs = rest[:len(_EDGE_W_NAMES)]
        ws_ref, we_ref = rest[len(_EDGE_W_NAMES):]
        w = {n: r[...] for n, r in zip(_EDGE_W_NAMES, wrefs)}
        a = hs_ref[...]
        b = he_ref[...]
        logit = _edge_mlp(a, b, w)
        s = 1.0 / (1.0 + jnp.exp(-logit))
        rid = (pl.program_id(0) * R_EDGE
               + lax.broadcasted_iota(jnp.int32, (R_EDGE, 1), 0))
        s = jnp.where(rid < E, s, 0.0)
        ws_ref[...] = s * a
        we_ref[...] = s * b

    wvals = [ew[n] for n in _EDGE_W_NAMES]
    return pl.pallas_call(
        body,
        grid=(EP // R_EDGE,),
        in_specs=[_row_spec(R_EDGE, D), _row_spec(R_EDGE, D)]
                 + [_full_spec(v.shape) for v in wvals],
        out_specs=[_row_spec(R_EDGE, D), _row_spec(R_EDGE, D)],
        out_shape=[jax.ShapeDtypeStruct((EP, D), jnp.float32),
                   jax.ShapeDtypeStruct((EP, D), jnp.float32)],
    )(hs, he, *wvals)


def _tc_edge_logits(hs, he, ew):
    def body(hs_ref, he_ref, *rest):
        wrefs = rest[:len(_EDGE_W_NAMES)]
        out_ref = rest[len(_EDGE_W_NAMES)]
        w = {n: r[...] for n, r in zip(_EDGE_W_NAMES, wrefs)}
        out_ref[...] = _edge_mlp(hs_ref[...], he_ref[...], w)

    wvals = [ew[n] for n in _EDGE_W_NAMES]
    return pl.pallas_call(
        body,
        grid=(EP // R_EDGE,),
        in_specs=[_row_spec(R_EDGE, D), _row_spec(R_EDGE, D)]
                 + [_full_spec(v.shape) for v in wvals],
        out_specs=_row_spec(R_EDGE, 1),
        out_shape=jax.ShapeDtypeStruct((EP, 1), jnp.float32),
    )(hs, he, *wvals)


_NODE_W_NAMES = ("Va", "Vb", "c1", "g1", "be1", "V2", "c2", "g2", "be2",
                 "V3", "c3")


def _tc_node(m0, m1, h, xpad, nw):
    def body(m0_ref, m1_ref, h_ref, xp_ref, *rest):
        wrefs = rest[:len(_NODE_W_NAMES)]
        out_ref = rest[len(_NODE_W_NAMES)]
        w = {n: r[...] for n, r in zip(_NODE_W_NAMES, wrefs)}
        m = m0_ref[...] + m1_ref[...]
        hh = h_ref[...]
        t = _ln_tanh(_dot(m, w["Va"]) + _dot(hh, w["Vb"]) + w["c1"],
                     w["g1"], w["be1"], HID)
        t = _ln_tanh(_dot(t, w["V2"]) + w["c2"], w["g2"], w["be2"], HID)
        t = _dot(t, w["V3"]) + w["c3"]
        out_ref[...] = hh + t + xp_ref[...]

    wvals = [nw[n] for n in _NODE_W_NAMES]
    return pl.pallas_call(
        body,
        grid=(N // R_NODE,),
        in_specs=[_row_spec(R_NODE, D)] * 4 + [_full_spec(v.shape) for v in wvals],
        out_specs=_row_spec(R_NODE, D),
        out_shape=jax.ShapeDtypeStruct((N, D), jnp.float32),
    )(m0, m1, h, xpad, *wvals)


_INP_W_NAMES = ("U1", "d1", "g1", "be1", "U2", "d2", "g2", "be2",
                "U3", "d3", "g3", "be3")


def _tc_input(x0, xpad, iw):
    def body(x0_ref, xp_ref, *rest):
        wrefs = rest[:len(_INP_W_NAMES)]
        out_ref = rest[len(_INP_W_NAMES)]
        w = {n: r[...] for n, r in zip(_INP_W_NAMES, wrefs)}
        t = _ln_tanh(_dot(x0_ref[...], w["U1"]) + w["d1"], w["g1"], w["be1"], HID)
        t = _ln_tanh(_dot(t, w["U2"]) + w["d2"], w["g2"], w["be2"], HID)
        t = _ln_tanh(_dot(t, w["U3"]) + w["d3"], w["g3"], w["be3"], HID)
        out_ref[...] = t + xp_ref[...]

    wvals = [iw[n] for n in _INP_W_NAMES]
    return pl.pallas_call(
        body,
        grid=(N // R_NODE,),
        in_specs=[_row_spec(R_NODE, D)] * 2 + [_full_spec(v.shape) for v in wvals],
        out_specs=_row_spec(R_NODE, D),
        out_shape=jax.ShapeDtypeStruct((N, D), jnp.float32),
    )(x0, xpad, *wvals)


def _pad2(W):
    return jnp.zeros((D, D), jnp.float32).at[:W.shape[0], :W.shape[1]].set(W)


def _pad1(v):
    return jnp.zeros((1, D), jnp.float32).at[0, :v.shape[0]].set(v)


def _prep_edge_weights(edge_params):
    (W1, b1, g1, be1), (W2, b2, g2, be2), (W3, b3, g3, be3), (W4, b4) = edge_params
    return {
        "Wa": _pad2(W1[:LAT]), "Wb": _pad2(W1[LAT:]),
        "b1": _pad1(b1), "g1": _pad1(g1), "be1": _pad1(be1),
        "W2": _pad2(W2), "b2": _pad1(b2), "g2": _pad1(g2), "be2": _pad1(be2),
        "W3": _pad2(W3), "b3": _pad1(b3), "g3": _pad1(g3), "be3": _pad1(be3),
        "w4": _pad1(W4[:, 0]), "b4": b4.reshape(1, 1).astype(jnp.float32),
    }


def _prep_node_weights(node_params):
    (V1, c1, g1, be1), (V2, c2, g2, be2), (V3, c3) = node_params
    return {
        "Va": _pad2(V1[:LAT]), "Vb": _pad2(V1[LAT:]),
        "c1": _pad1(c1), "g1": _pad1(g1), "be1": _pad1(be1),
        "V2": _pad2(V2), "c2": _pad1(c2), "g2": _pad1(g2), "be2": _pad1(be2),
        "V3": _pad2(V3), "c3": _pad1(c3),
    }


def _prep_inp_weights(inp_params):
    (U1, d1, g1, be1), (U2, d2, g2, be2), (U3, d3, g3, be3) = inp_params
    return {
        "U1": _pad2(U1), "d1": _pad1(d1), "g1": _pad1(g1), "be1": _pad1(be1),
        "U2": _pad2(U2), "d2": _pad1(d2), "g2": _pad1(g2), "be2": _pad1(be2),
        "U3": _pad2(U3), "d3": _pad1(d3), "g3": _pad1(g3), "be3": _pad1(be3),
    }


def kernel(x, edge_index, inp_params, edge_params, node_params):
    start = edge_index[0].astype(jnp.int32)
    end = edge_index[1].astype(jnp.int32)
    s2d = jnp.zeros((EP,), jnp.int32).at[:E].set(start).reshape(NW * CHUNKS, CH)
    e2d = jnp.zeros((EP,), jnp.int32).at[:E].set(end).reshape(NW * CHUNKS, CH)

    xpad = jnp.zeros((N, D), jnp.float32).at[:, HID:LAT].set(x)
    x0 = jnp.zeros((N, D), jnp.float32).at[:, :x.shape[1]].set(x)
    zeros_nd = jnp.zeros((N, D), jnp.float32)

    iw = _prep_inp_weights(inp_params)
    ew = _prep_edge_weights(edge_params)
    nw = _prep_node_weights(node_params)

    h = _tc_input(x0, xpad, iw)
    for _ in range(8):
        hs, he = _sc_gather(h, s2d, e2d)
        ws, we = _tc_edge_messages(hs, he, ew)
        mp = _sc_scatter(ws, we, s2d, e2d, zeros_nd)
        h = _tc_node(mp[0], mp[1], h, xpad, nw)

    hs, he = _sc_gather(h, s2d, e2d)
    logits = _tc_edge_logits(hs, he, ew)
    return logits[:E, 0]
